# baseline (speedup 1.0000x reference)
import functools

import jax
import jax.numpy as jnp
from jax import lax
from jax.experimental import pallas as pl
from jax.experimental.pallas import tpu as pltpu
from jax.experimental.pallas import tpu_sc as plsc

EPS = 1e-15
D = 128
C = 10
CP = C - 1
NC, NS = 2, 16
NW = NC * NS
R = 40
IDXC = R * C


@functools.cache
def _make_sc_dots(b_total: int):
    rows_w = b_total // NW
    ch = rows_w // R
    assert rows_w * NW == b_total and ch * R == rows_w
    orows = R * 16 // D

    mesh = plsc.VectorSubcoreMesh(core_axis_name="c", subcore_axis_name="s")

    nhalf = ch // 2
    assert nhalf * 2 == ch

    @functools.partial(
        pl.kernel,
        mesh=mesh,
        compiler_params=pltpu.CompilerParams(needs_layout_passes=False),
        out_type=jax.ShapeDtypeStruct((b_total, 16), jnp.float32),
        scratch_types=[
            pltpu.VMEM((rows_w * C,), jnp.int32),
            pltpu.VMEM((IDXC, D), jnp.float32),
            pltpu.VMEM((IDXC, D), jnp.float32),
            pltpu.VMEM((R, 16), jnp.float32),
            pltpu.VMEM((R, 16), jnp.float32),
            pltpu.SemaphoreType.DMA,
            pltpu.SemaphoreType.DMA,
            pltpu.SemaphoreType.DMA,
            pltpu.SemaphoreType.DMA,
        ],
    )
    def sc_dots(idx_hbm, emb_hbm, out_hbm, idx_v, rows_a, rows_b,
                dots_a, dots_b, sem_a, sem_b, osem_a, osem_b):
        w = lax.axis_index("s") * NC + lax.axis_index("c")
        for j in range(C):
            pltpu.async_copy(
                idx_hbm.at[pl.ds(j * b_total + w * rows_w, rows_w)],
                idx_v.at[pl.ds(j * rows_w, rows_w)],
                sem_a,
            )
        for j in range(C):
            pltpu.make_async_copy(
                idx_hbm.at[pl.ds(j * b_total, rows_w)],
                idx_v.at[pl.ds(j * rows_w, rows_w)],
                sem_a,
            ).wait()
        lane = lax.iota(jnp.int32, 16)
        lane_eq = [lane == jj for jj in range(CP)]

        def fire(k, rows_v, sem):
            for j in range(C):
                pltpu.async_copy(
                    emb_hbm.at[idx_v.at[pl.ds(j * rows_w + k * R, R)]],
                    rows_v.at[pl.ds(j * R, R)],
                    sem,
                )

        def drain(rows_v, sem):
            for j in range(C):
                pltpu.make_async_copy(
                    emb_hbm.at[idx_v.at[pl.ds(j * rows_w, R)]],
                    rows_v.at[pl.ds(j * R, R)],
                    sem,
                ).wait()

        def out_slice(k):
            return out_hbm.at[pl.ds((w * ch + k) * R, R)]

        def compute(rows_v, dots_v):
            @plsc.parallel_loop(0, R, unroll=8)
            def row_body(r):
                s = [rows_v[r, pl.ds(cc * 16, 16)] for cc in range(8)]
                coll = jnp.zeros((16,), jnp.float32)
                for j in range(1, C):
                    p = [
                        s[cc] * rows_v[j * R + r, pl.ds(cc * 16, 16)]
                        for cc in range(8)
                    ]
                    q = [p[0] + p[1], p[2] + p[3], p[4] + p[5], p[6] + p[7]]
                    acc = (q[0] + q[1]) + (q[2] + q[3])
                    coll = jnp.where(lane_eq[j - 1], jnp.sum(acc), coll)
                dots_v[r, pl.ds(0, 16)] = coll

        def owait(k, dots_v, osem):
            pltpu.make_async_copy(dots_v, out_slice(k), osem).wait()

        fire(0, rows_a, sem_a)

        def body(i, carry):
            k0 = 2 * i
            k1 = k0 + 1
            fire(k1, rows_b, sem_b)
            drain(rows_a, sem_a)

            @pl.when(i > 0)
            def _():
                owait(k0, dots_a, osem_a)

            compute(rows_a, dots_a)
            pltpu.async_copy(dots_a, out_slice(k0), osem_a)

            @pl.when(i < nhalf - 1)
            def _():
                fire(k0 + 2, rows_a, sem_a)

            drain(rows_b, sem_b)

            @pl.when(i > 0)
            def _():
                owait(k1, dots_b, osem_b)

            compute(rows_b, dots_b)
            pltpu.async_copy(dots_b, out_slice(k1), osem_b)
            return carry

        lax.fori_loop(0, nhalf, body, 0)
        owait(ch - 2, dots_a, osem_a)
        owait(ch - 1, dots_b, osem_b)

    return sc_dots


@functools.cache
def _make_loss_tc(rows: int, pos_rows: int, denom: float):
    def body(d_ref, o_ref):
        x = d_ref[:]
        sig = 1.0 / (1.0 + jnp.exp(-x))
        lane16 = lax.broadcasted_iota(jnp.int32, x.shape, 1) % 16
        rid = lax.broadcasted_iota(jnp.int32, x.shape, 0)
        val = jnp.where(
            rid < pos_rows, -jnp.log(sig + EPS), -jnp.log(1.0 - sig + EPS)
        )
        val = jnp.where(lane16 < CP, val, 0.0)
        o_ref[:, :] = jnp.reshape(jnp.sum(val) / denom, (1, 1))

    return pl.pallas_call(
        body, out_shape=jax.ShapeDtypeStruct((1, 1), jnp.float32)
    )


def kernel(pos_rw, neg_rw, emb):
    b = pos_rw.shape[0]
    idx_t = jnp.concatenate(
        [pos_rw, neg_rw], axis=0).T.reshape(-1).astype(jnp.int32)
    dots = _make_sc_dots(2 * b)(idx_t, emb)
    dots2d = dots.reshape(-1, D)
    pos_rows = b * 16 // D
    loss = _make_loss_tc(dots2d.shape[0], pos_rows, float(b * CP))(dots2d)
    return loss[0, 0]


# speedup vs baseline: 1.0695x; 1.0695x over previous
import functools

import jax
import jax.numpy as jnp
from jax import lax
from jax.experimental import pallas as pl
from jax.experimental.pallas import tpu as pltpu
from jax.experimental.pallas import tpu_sc as plsc

EPS = 1e-15
D = 128
C = 10
CP = C - 1
NC, NS = 2, 16
NW = NC * NS
R = 40
IDXC = R * C


@functools.cache
def _make_sc_dots(b_total: int):
    rows_w = b_total // NW
    ch = rows_w // R
    assert rows_w * NW == b_total and ch * R == rows_w
    orows = R * 16 // D

    mesh = plsc.VectorSubcoreMesh(core_axis_name="c", subcore_axis_name="s")

    nhalf = ch // 2
    assert nhalf * 2 == ch

    @functools.partial(
        pl.kernel,
        mesh=mesh,
        compiler_params=pltpu.CompilerParams(needs_layout_passes=False),
        out_type=jax.ShapeDtypeStruct((b_total, 16), jnp.float32),
        scratch_types=[
            pltpu.VMEM((rows_w * C,), jnp.int32),
            pltpu.VMEM((IDXC, D), jnp.float32),
            pltpu.VMEM((IDXC, D), jnp.float32),
            pltpu.VMEM((R, 16), jnp.float32),
            pltpu.VMEM((R, 16), jnp.float32),
            pltpu.SemaphoreType.DMA,
            pltpu.SemaphoreType.DMA,
            pltpu.SemaphoreType.DMA,
            pltpu.SemaphoreType.DMA,
        ],
    )
    def sc_dots(idx_hbm, emb_hbm, out_hbm, idx_v, rows_a, rows_b,
                dots_a, dots_b, sem_a, sem_b, osem_a, osem_b):
        w = lax.axis_index("s") * NC + lax.axis_index("c")
        for j in range(C):
            pltpu.async_copy(
                idx_hbm.at[pl.ds(j * b_total + w * rows_w, rows_w)],
                idx_v.at[pl.ds(j * rows_w, rows_w)],
                sem_a,
            )
        for j in range(C):
            pltpu.make_async_copy(
                idx_hbm.at[pl.ds(j * b_total, rows_w)],
                idx_v.at[pl.ds(j * rows_w, rows_w)],
                sem_a,
            ).wait()
        lane = lax.iota(jnp.int32, 16)
        lane_eq = [lane == jj for jj in range(CP)]

        def fire(k, rows_v, sem):
            for j in range(C):
                pltpu.async_copy(
                    emb_hbm.at[idx_v.at[pl.ds(j * rows_w + k * R, R)]],
                    rows_v.at[pl.ds(j * R, R)],
                    sem,
                )

        def drain(rows_v, sem):
            for j in range(C):
                pltpu.make_async_copy(
                    emb_hbm.at[idx_v.at[pl.ds(j * rows_w, R)]],
                    rows_v.at[pl.ds(j * R, R)],
                    sem,
                ).wait()

        def out_slice(k):
            return out_hbm.at[pl.ds((w * ch + k) * R, R)]

        def compute(rows_v, dots_v):
            @plsc.parallel_loop(0, R, unroll=4)
            def row_body(r):
                s = [rows_v[r, pl.ds(cc * 16, 16)] for cc in range(8)]
                coll = jnp.zeros((16,), jnp.float32)
                for j in range(1, C):
                    p = [
                        s[cc] * rows_v[j * R + r, pl.ds(cc * 16, 16)]
                        for cc in range(8)
                    ]
                    q = [p[0] + p[1], p[2] + p[3], p[4] + p[5], p[6] + p[7]]
                    acc = (q[0] + q[1]) + (q[2] + q[3])
                    coll = jnp.where(lane_eq[j - 1], jnp.sum(acc), coll)
                dots_v[r, pl.ds(0, 16)] = coll

        def owait(k, dots_v, osem):
            pltpu.make_async_copy(dots_v, out_slice(k), osem).wait()

        fire(0, rows_a, sem_a)

        def body(i, carry):
            k0 = 2 * i
            k1 = k0 + 1
            fire(k1, rows_b, sem_b)
            drain(rows_a, sem_a)

            @pl.when(i > 0)
            def _():
                owait(k0, dots_a, osem_a)

            compute(rows_a, dots_a)
            pltpu.async_copy(dots_a, out_slice(k0), osem_a)

            @pl.when(i < nhalf - 1)
            def _():
                fire(k0 + 2, rows_a, sem_a)

            drain(rows_b, sem_b)

            @pl.when(i > 0)
            def _():
                owait(k1, dots_b, osem_b)

            compute(rows_b, dots_b)
            pltpu.async_copy(dots_b, out_slice(k1), osem_b)
            return carry

        lax.fori_loop(0, nhalf, body, 0)
        owait(ch - 2, dots_a, osem_a)
        owait(ch - 1, dots_b, osem_b)

    return sc_dots


@functools.cache
def _make_loss_tc(rows: int, pos_rows: int, denom: float):
    def body(d_ref, o_ref):
        x = d_ref[:]
        sig = 1.0 / (1.0 + jnp.exp(-x))
        lane16 = lax.broadcasted_iota(jnp.int32, x.shape, 1) % 16
        rid = lax.broadcasted_iota(jnp.int32, x.shape, 0)
        val = jnp.where(
            rid < pos_rows, -jnp.log(sig + EPS), -jnp.log(1.0 - sig + EPS)
        )
        val = jnp.where(lane16 < CP, val, 0.0)
        o_ref[:, :] = jnp.reshape(jnp.sum(val) / denom, (1, 1))

    return pl.pallas_call(
        body, out_shape=jax.ShapeDtypeStruct((1, 1), jnp.float32)
    )


def kernel(pos_rw, neg_rw, emb):
    b = pos_rw.shape[0]
    idx_t = jnp.concatenate(
        [pos_rw, neg_rw], axis=0).T.reshape(-1).astype(jnp.int32)
    dots = _make_sc_dots(2 * b)(idx_t, emb)
    dots2d = dots.reshape(-1, D)
    pos_rows = b * 16 // D
    loss = _make_loss_tc(dots2d.shape[0], pos_rows, float(b * CP))(dots2d)
    return loss[0, 0]


# speedup vs baseline: 1.2510x; 1.1697x over previous
import functools

import jax
import jax.numpy as jnp
from jax import lax
from jax.experimental import pallas as pl
from jax.experimental.pallas import tpu as pltpu
from jax.experimental.pallas import tpu_sc as plsc

EPS = 1e-15
D = 128
C = 10
CP = C - 1
NC, NS = 2, 16
NW = NC * NS
R = 40
IDXC = R * C


@functools.cache
def _make_sc_dots(b_total: int):
    rows_w = b_total // NW
    ch = rows_w // R
    assert rows_w * NW == b_total and ch * R == rows_w
    orows = R * 16 // D

    mesh = plsc.VectorSubcoreMesh(core_axis_name="c", subcore_axis_name="s")

    nhalf = ch // 2
    assert nhalf * 2 == ch

    @functools.partial(
        pl.kernel,
        mesh=mesh,
        compiler_params=pltpu.CompilerParams(needs_layout_passes=False),
        out_type=jax.ShapeDtypeStruct((b_total, 16), jnp.float32),
        scratch_types=[
            pltpu.VMEM((rows_w * C,), jnp.int32),
            pltpu.VMEM((IDXC, D), jnp.float32),
            pltpu.VMEM((IDXC, D), jnp.float32),
            pltpu.VMEM((R, 16), jnp.float32),
            pltpu.VMEM((R, 16), jnp.float32),
            pltpu.SemaphoreType.DMA,
            pltpu.SemaphoreType.DMA,
            pltpu.SemaphoreType.DMA,
            pltpu.SemaphoreType.DMA,
        ],
    )
    def sc_dots(idx_hbm, emb_hbm, out_hbm, idx_v, rows_a, rows_b,
                dots_a, dots_b, sem_a, sem_b, osem_a, osem_b):
        w = lax.axis_index("s") * NC + lax.axis_index("c")
        for j in range(C):
            pltpu.async_copy(
                idx_hbm.at[pl.ds(j * b_total + w * rows_w, rows_w)],
                idx_v.at[pl.ds(j * rows_w, rows_w)],
                sem_a,
            )
        for j in range(C):
            pltpu.make_async_copy(
                idx_hbm.at[pl.ds(j * b_total, rows_w)],
                idx_v.at[pl.ds(j * rows_w, rows_w)],
                sem_a,
            ).wait()
        lane = lax.iota(jnp.int32, 16)
        lane_eq = [lane == jj for jj in range(CP)]

        def fire(k, rows_v, sem):
            for j in range(C):
                pltpu.async_copy(
                    emb_hbm.at[idx_v.at[pl.ds(j * rows_w + k * R, R)]],
                    rows_v.at[pl.ds(j * R, R)],
                    sem,
                )

        def drain(rows_v, sem):
            for j in range(C):
                pltpu.make_async_copy(
                    emb_hbm.at[idx_v.at[pl.ds(j * rows_w, R)]],
                    rows_v.at[pl.ds(j * R, R)],
                    sem,
                ).wait()

        def out_slice(k):
            return out_hbm.at[pl.ds((w * ch + k) * R, R)]

        def compute(rows_v, dots_v):
            @plsc.parallel_loop(0, R, unroll=2)
            def row_body(r):
                s = [rows_v[r, pl.ds(cc * 16, 16)] for cc in range(8)]
                coll = jnp.zeros((16,), jnp.float32)
                for j in range(1, C):
                    p = [
                        s[cc] * rows_v[j * R + r, pl.ds(cc * 16, 16)]
                        for cc in range(8)
                    ]
                    q = [p[0] + p[1], p[2] + p[3], p[4] + p[5], p[6] + p[7]]
                    acc = (q[0] + q[1]) + (q[2] + q[3])
                    coll = jnp.where(lane_eq[j - 1], jnp.sum(acc), coll)
                dots_v[r, pl.ds(0, 16)] = coll

        def owait(k, dots_v, osem):
            pltpu.make_async_copy(dots_v, out_slice(k), osem).wait()

        fire(0, rows_a, sem_a)

        def body(i, carry):
            k0 = 2 * i
            k1 = k0 + 1
            fire(k1, rows_b, sem_b)
            drain(rows_a, sem_a)

            @pl.when(i > 0)
            def _():
                owait(k0, dots_a, osem_a)

            compute(rows_a, dots_a)
            pltpu.async_copy(dots_a, out_slice(k0), osem_a)

            @pl.when(i < nhalf - 1)
            def _():
                fire(k0 + 2, rows_a, sem_a)

            drain(rows_b, sem_b)

            @pl.when(i > 0)
            def _():
                owait(k1, dots_b, osem_b)

            compute(rows_b, dots_b)
            pltpu.async_copy(dots_b, out_slice(k1), osem_b)
            return carry

        lax.fori_loop(0, nhalf, body, 0)
        owait(ch - 2, dots_a, osem_a)
        owait(ch - 1, dots_b, osem_b)

    return sc_dots


@functools.cache
def _make_loss_tc(rows: int, pos_rows: int, denom: float):
    def body(d_ref, o_ref):
        x = d_ref[:]
        sig = 1.0 / (1.0 + jnp.exp(-x))
        lane16 = lax.broadcasted_iota(jnp.int32, x.shape, 1) % 16
        rid = lax.broadcasted_iota(jnp.int32, x.shape, 0)
        val = jnp.where(
            rid < pos_rows, -jnp.log(sig + EPS), -jnp.log(1.0 - sig + EPS)
        )
        val = jnp.where(lane16 < CP, val, 0.0)
        o_ref[:, :] = jnp.reshape(jnp.sum(val) / denom, (1, 1))

    return pl.pallas_call(
        body, out_shape=jax.ShapeDtypeStruct((1, 1), jnp.float32)
    )


def kernel(pos_rw, neg_rw, emb):
    b = pos_rw.shape[0]
    idx_t = jnp.concatenate(
        [pos_rw, neg_rw], axis=0).T.reshape(-1).astype(jnp.int32)
    dots = _make_sc_dots(2 * b)(idx_t, emb)
    dots2d = dots.reshape(-1, D)
    pos_rows = b * 16 // D
    loss = _make_loss_tc(dots2d.shape[0], pos_rows, float(b * CP))(dots2d)
    return loss[0, 0]


# speedup vs baseline: 1.2625x; 1.0092x over previous
import functools

import jax
import jax.numpy as jnp
from jax import lax
from jax.experimental import pallas as pl
from jax.experimental.pallas import tpu as pltpu
from jax.experimental.pallas import tpu_sc as plsc

EPS = 1e-15
D = 128
C = 10
CP = C - 1
NC, NS = 2, 16
NW = NC * NS
R = 40
IDXC = R * C


@functools.cache
def _make_sc_dots(b_total: int):
    rows_w = b_total // NW
    ch = rows_w // R
    assert rows_w * NW == b_total and ch * R == rows_w
    orows = R * 16 // D

    mesh = plsc.VectorSubcoreMesh(core_axis_name="c", subcore_axis_name="s")

    nhalf = ch // 2
    assert nhalf * 2 == ch

    @functools.partial(
        pl.kernel,
        mesh=mesh,
        compiler_params=pltpu.CompilerParams(needs_layout_passes=False),
        out_type=jax.ShapeDtypeStruct((b_total, 16), jnp.float32),
        scratch_types=[
            pltpu.VMEM((rows_w * C,), jnp.int32),
            pltpu.VMEM((IDXC, D), jnp.float32),
            pltpu.VMEM((IDXC, D), jnp.float32),
            pltpu.VMEM((R, 16), jnp.float32),
            pltpu.VMEM((R, 16), jnp.float32),
            pltpu.SemaphoreType.DMA,
            pltpu.SemaphoreType.DMA,
            pltpu.SemaphoreType.DMA,
            pltpu.SemaphoreType.DMA,
        ],
    )
    def sc_dots(idx_hbm, emb_hbm, out_hbm, idx_v, rows_a, rows_b,
                dots_a, dots_b, sem_a, sem_b, osem_a, osem_b):
        w = lax.axis_index("s") * NC + lax.axis_index("c")
        for j in range(C):
            pltpu.async_copy(
                idx_hbm.at[pl.ds(j * b_total + w * rows_w, rows_w)],
                idx_v.at[pl.ds(j * rows_w, rows_w)],
                sem_a,
            )
        for j in range(C):
            pltpu.make_async_copy(
                idx_hbm.at[pl.ds(j * b_total, rows_w)],
                idx_v.at[pl.ds(j * rows_w, rows_w)],
                sem_a,
            ).wait()
        lane = lax.iota(jnp.int32, 16)
        lane_eq = [lane == jj for jj in range(CP)]

        def fire(k, rows_v, sem):
            for j in range(C):
                pltpu.async_copy(
                    emb_hbm.at[idx_v.at[pl.ds(j * rows_w + k * R, R)]],
                    rows_v.at[pl.ds(j * R, R)],
                    sem,
                )

        def drain(rows_v, sem):
            for j in range(C):
                pltpu.make_async_copy(
                    emb_hbm.at[idx_v.at[pl.ds(j * rows_w, R)]],
                    rows_v.at[pl.ds(j * R, R)],
                    sem,
                ).wait()

        def out_slice(k):
            return out_hbm.at[pl.ds((w * ch + k) * R, R)]

        def compute(rows_v, dots_v):
            @plsc.parallel_loop(0, R, unroll=1)
            def row_body(r):
                s = [rows_v[r, pl.ds(cc * 16, 16)] for cc in range(8)]
                coll = jnp.zeros((16,), jnp.float32)
                for j in range(1, C):
                    p = [
                        s[cc] * rows_v[j * R + r, pl.ds(cc * 16, 16)]
                        for cc in range(8)
                    ]
                    q = [p[0] + p[1], p[2] + p[3], p[4] + p[5], p[6] + p[7]]
                    acc = (q[0] + q[1]) + (q[2] + q[3])
                    coll = jnp.where(lane_eq[j - 1], jnp.sum(acc), coll)
                dots_v[r, pl.ds(0, 16)] = coll

        def owait(k, dots_v, osem):
            pltpu.make_async_copy(dots_v, out_slice(k), osem).wait()

        fire(0, rows_a, sem_a)

        def body(i, carry):
            k0 = 2 * i
            k1 = k0 + 1
            fire(k1, rows_b, sem_b)
            drain(rows_a, sem_a)

            @pl.when(i > 0)
            def _():
                owait(k0, dots_a, osem_a)

            compute(rows_a, dots_a)
            pltpu.async_copy(dots_a, out_slice(k0), osem_a)

            @pl.when(i < nhalf - 1)
            def _():
                fire(k0 + 2, rows_a, sem_a)

            drain(rows_b, sem_b)

            @pl.when(i > 0)
            def _():
                owait(k1, dots_b, osem_b)

            compute(rows_b, dots_b)
            pltpu.async_copy(dots_b, out_slice(k1), osem_b)
            return carry

        lax.fori_loop(0, nhalf, body, 0)
        owait(ch - 2, dots_a, osem_a)
        owait(ch - 1, dots_b, osem_b)

    return sc_dots


@functools.cache
def _make_loss_tc(rows: int, pos_rows: int, denom: float):
    def body(d_ref, o_ref):
        x = d_ref[:]
        sig = 1.0 / (1.0 + jnp.exp(-x))
        lane16 = lax.broadcasted_iota(jnp.int32, x.shape, 1) % 16
        rid = lax.broadcasted_iota(jnp.int32, x.shape, 0)
        val = jnp.where(
            rid < pos_rows, -jnp.log(sig + EPS), -jnp.log(1.0 - sig + EPS)
        )
        val = jnp.where(lane16 < CP, val, 0.0)
        o_ref[:, :] = jnp.reshape(jnp.sum(val) / denom, (1, 1))

    return pl.pallas_call(
        body, out_shape=jax.ShapeDtypeStruct((1, 1), jnp.float32)
    )


def kernel(pos_rw, neg_rw, emb):
    b = pos_rw.shape[0]
    idx_t = jnp.concatenate(
        [pos_rw, neg_rw], axis=0).T.reshape(-1).astype(jnp.int32)
    dots = _make_sc_dots(2 * b)(idx_t, emb)
    dots2d = dots.reshape(-1, D)
    pos_rows = b * 16 // D
    loss = _make_loss_tc(dots2d.shape[0], pos_rows, float(b * CP))(dots2d)
    return loss[0, 0]
